# baseline (speedup 1.0000x reference)
import functools

import jax
import jax.numpy as jnp
from jax import lax
from jax.experimental import pallas as pl
from jax.experimental.pallas import tpu as pltpu
from jax.experimental.pallas import tpu_sc as plsc

EMBED_DIM = 128
L_WORD = 50
L_RELA = 8
WPAD = 100
EPS = 1e-8
LANES = 16
NCHUNK = EMBED_DIM // LANES
BPW = 128
RELA_VOCAB = 1000


def _pair_row(buf, r):
    out = []
    for c in range(NCHUNK // 2):
        x = buf[r, pl.ds(c * LANES, LANES)]
        out.append(lax.bitcast_convert_type(x << 16, jnp.float32))
        out.append(lax.bitcast_convert_type(x & jnp.int32(-65536), jnp.float32))
    return out


def _rowsum_pair(wbuf, n):
    zero = jnp.zeros((LANES,), jnp.float32)

    def body(r, carry):
        qa, ra = carry
        qrow = _pair_row(wbuf, r)
        rrow = _pair_row(wbuf, r + n)
        qa = tuple(qa[j] + qrow[j] for j in range(NCHUNK))
        ra = tuple(ra[j] + rrow[j] for j in range(NCHUNK))
        return (qa, ra)

    init = (tuple(zero for _ in range(NCHUNK)), tuple(zero for _ in range(NCHUNK)))
    return lax.fori_loop(0, n, body, init, unroll=2)


def _sc_body(qidx_hbm, tidx_hbm, ridx_hbm, wtab, rtab, out_hbm,
             qidx_v, tidx_v, ridx_v, rtab_v, wb0, wb1, score_v,
             swa, swb, sta, stb, srt):
    nc = 2
    wid = lax.axis_index("s") * nc + lax.axis_index("c")
    base = wid * BPW

    pltpu.sync_copy(qidx_hbm.at[pl.ds(base, BPW)], qidx_v)
    pltpu.sync_copy(tidx_hbm.at[pl.ds(base, BPW)], tidx_v)
    pltpu.sync_copy(ridx_hbm.at[pl.ds(base * L_RELA, BPW * L_RELA)],
                    ridx_v.at[pl.ds(0, BPW * L_RELA)])
    pltpu.async_copy(rtab, rtab_v, srt)

    wbufs = (wb0, wb1)
    sws = (swa, swb)
    sts = (sta, stb)

    def issue_w(e, b):
        pltpu.async_copy(wtab.at[qidx_v.at[e]],
                         wbufs[b].at[pl.ds(0, L_WORD)], sws[b])
        pltpu.async_copy(wtab.at[tidx_v.at[e]],
                         wbufs[b].at[pl.ds(L_WORD, L_WORD)], sts[b])

    def wait_w(e, b):
        pltpu.make_async_copy(wtab.at[qidx_v.at[e]],
                              wbufs[b].at[pl.ds(0, L_WORD)], sws[b]).wait()
        pltpu.make_async_copy(wtab.at[tidx_v.at[e]],
                              wbufs[b].at[pl.ds(L_WORD, L_WORD)], sts[b]).wait()

    issue_w(0, 0)
    pltpu.make_async_copy(rtab, rtab_v, srt).wait()

    lane_iota = lax.iota(jnp.int32, LANES)

    def rela_sum(e):
        acc = [jnp.zeros((LANES,), jnp.float32) for _ in range(NCHUNK)]
        rv = ridx_v[pl.ds(e * L_RELA, LANES)]
        for i in range(L_RELA):
            row = _pair_row(rtab_v, rv[i])
            for j in range(NCHUNK):
                acc[j] = acc[j] + row[j]
        return acc

    def compute_elem(wbuf, e):
        qsum, rtsum = _rowsum_pair(wbuf, L_WORD)
        rsum = rela_sum(e)
        dv = jnp.zeros((LANES,), jnp.float32)
        n1 = jnp.zeros((LANES,), jnp.float32)
        n2 = jnp.zeros((LANES,), jnp.float32)
        for j in range(NCHUNK):
            q = qsum[j] * (1.0 / L_WORD)
            rm = rtsum[j] * (1.0 / L_WORD) + rsum[j] * (1.0 / L_RELA)
            dv = dv + q * rm
            n1 = n1 + q * q
            n2 = n2 + rm * rm
        for s in (8, 4, 2, 1):
            idx = lane_iota ^ s
            dv = dv + dv.at[idx].get(mode="promise_in_bounds")
            n1 = n1 + n1.at[idx].get(mode="promise_in_bounds")
            n2 = n2 + n2.at[idx].get(mode="promise_in_bounds")
        return dv, n1, n2

    @pl.loop(0, BPW // LANES)
    def _group(g):
        def pbody(p, carry):
            dacc, n1acc, n2acc = carry
            for b in range(2):
                k = p * 2 + b
                e = g * LANES + k
                nxt = e + 1

                @pl.when(nxt < BPW)
                def _():
                    issue_w(jnp.minimum(nxt, BPW - 1), 1 - b)

                wait_w(e, b)
                d_v, n1_v, n2_v = compute_elem(wbufs[b], e)
                sel = lane_iota == k
                dacc = jnp.where(sel, d_v, dacc)
                n1acc = jnp.where(sel, n1_v, n1acc)
                n2acc = jnp.where(sel, n2_v, n2acc)
            return (dacc, n1acc, n2acc)

        zero = jnp.zeros((LANES,), jnp.float32)
        dacc, n1acc, n2acc = lax.fori_loop(0, LANES // 2, pbody,
                                           (zero, zero, zero))

        prod = n1acc * n2acc
        yi = (lax.bitcast_convert_type(prod, jnp.int32) >> 1) \
            + jnp.int32(0x1FBD1DF5)
        y = lax.bitcast_convert_type(yi, jnp.float32)
        for _ in range(3):
            y = 0.5 * (y + prod / y)
        score = dacc / jnp.maximum(y, EPS)
        score_v[pl.ds(g * LANES, LANES)] = score

    pltpu.sync_copy(score_v, out_hbm.at[pl.ds(base, BPW)])


@functools.cache
def _build(batch):
    mesh = plsc.VectorSubcoreMesh(core_axis_name="c", subcore_axis_name="s")
    return pl.kernel(
        _sc_body,
        out_type=jax.ShapeDtypeStruct((batch,), jnp.float32),
        mesh=mesh,
        compiler_params=pltpu.CompilerParams(use_tc_tiling_on_sc=False),
        scratch_types=[
            pltpu.VMEM((BPW, L_WORD), jnp.int32),
            pltpu.VMEM((BPW, L_WORD), jnp.int32),
            pltpu.VMEM((BPW * L_RELA + LANES,), jnp.int32),
            pltpu.VMEM((RELA_VOCAB, EMBED_DIM // 2), jnp.int32),
            pltpu.VMEM((WPAD, EMBED_DIM // 2), jnp.int32),
            pltpu.VMEM((WPAD, EMBED_DIM // 2), jnp.int32),
            pltpu.VMEM((BPW,), jnp.float32),
            pltpu.SemaphoreType.DMA,
            pltpu.SemaphoreType.DMA,
            pltpu.SemaphoreType.DMA,
            pltpu.SemaphoreType.DMA,
            pltpu.SemaphoreType.DMA,
        ],
    )


def kernel(ques_x, rela_text_x, rela_x, word_emb, rela_emb):
    batch = ques_x.shape[0]
    qidx = ques_x.astype(jnp.int32)
    tidx = rela_text_x.astype(jnp.int32)
    ridx = rela_x.astype(jnp.int32).reshape(-1)
    def pack(tab):
        x = lax.bitcast_convert_type(tab, jnp.int32) + jnp.int32(0x8000)
        lo = lax.shift_right_logical(x[:, 0::2], 16)
        hi = x[:, 1::2] & jnp.int32(-65536)
        return lo | hi

    return _build(batch)(qidx, tidx, ridx, pack(word_emb), pack(rela_emb))


# speedup vs baseline: 8.4650x; 8.4650x over previous
import functools

import jax
import jax.numpy as jnp
from jax import lax
from jax.experimental import pallas as pl
from jax.experimental.pallas import tpu as pltpu
from jax.experimental.pallas import tpu_sc as plsc

EMBED_DIM = 128
L_WORD = 50
L_RELA = 8
WPAD = 100
EPS = 1e-8
LANES = 16
NCHUNK = EMBED_DIM // LANES
BPW = 128
RELA_VOCAB = 1000


def _pair_row(buf, r):
    out = []
    for c in range(NCHUNK // 2):
        x = buf[r, pl.ds(c * LANES, LANES)]
        out.append(lax.bitcast_convert_type(x << 16, jnp.float32))
        out.append(lax.bitcast_convert_type(x & jnp.int32(-65536), jnp.float32))
    return out


def _rowsum_pair(wbuf, n):
    zero = jnp.zeros((LANES,), jnp.float32)

    def body(r, carry):
        qa, ra = carry
        qrow = _pair_row(wbuf, r)
        rrow = _pair_row(wbuf, r + n)
        qa = tuple(qa[j] + qrow[j] for j in range(NCHUNK))
        ra = tuple(ra[j] + rrow[j] for j in range(NCHUNK))
        return (qa, ra)

    init = (tuple(zero for _ in range(NCHUNK)), tuple(zero for _ in range(NCHUNK)))
    return lax.fori_loop(0, n, body, init, unroll=2)


def _sc_body(qidx_hbm, tidx_hbm, ridx_hbm, wtab, rtab, out_hbm,
             qidx_v, tidx_v, ridx_v, rtab_v, wb0, wb1, score_v,
             swa, swb, sta, stb, srt):
    nc = 2
    wid = lax.axis_index("s") * nc + lax.axis_index("c")
    base = wid * BPW

    pltpu.sync_copy(qidx_hbm.at[pl.ds(base, BPW)], qidx_v)
    pltpu.sync_copy(tidx_hbm.at[pl.ds(base, BPW)], tidx_v)
    pltpu.sync_copy(ridx_hbm.at[pl.ds(base * L_RELA, BPW * L_RELA)],
                    ridx_v.at[pl.ds(0, BPW * L_RELA)])
    pltpu.async_copy(rtab, rtab_v, srt)

    wbufs = (wb0, wb1)
    sws = (swa, swb)
    sts = (sta, stb)

    def issue_w(e, b):
        pltpu.async_copy(wtab.at[qidx_v.at[e]],
                         wbufs[b].at[pl.ds(0, L_WORD)], sws[b])
        pltpu.async_copy(wtab.at[tidx_v.at[e]],
                         wbufs[b].at[pl.ds(L_WORD, L_WORD)], sts[b])

    def wait_w(e, b):
        pltpu.make_async_copy(wtab.at[qidx_v.at[e]],
                              wbufs[b].at[pl.ds(0, L_WORD)], sws[b]).wait()
        pltpu.make_async_copy(wtab.at[tidx_v.at[e]],
                              wbufs[b].at[pl.ds(L_WORD, L_WORD)], sts[b]).wait()

    issue_w(0, 0)
    pltpu.make_async_copy(rtab, rtab_v, srt).wait()

    lane_iota = lax.iota(jnp.int32, LANES)

    def rela_sum(e):
        acc = [jnp.zeros((LANES,), jnp.float32) for _ in range(NCHUNK)]
        rv = ridx_v[pl.ds(e * L_RELA, LANES)]
        for i in range(L_RELA):
            row = _pair_row(rtab_v, rv[i])
            for j in range(NCHUNK):
                acc[j] = acc[j] + row[j]
        return acc

    def compute_elem(wbuf, e):
        qsum, rtsum = _rowsum_pair(wbuf, L_WORD)
        rsum = rela_sum(e)
        dv = jnp.zeros((LANES,), jnp.float32)
        n1 = jnp.zeros((LANES,), jnp.float32)
        n2 = jnp.zeros((LANES,), jnp.float32)
        for j in range(NCHUNK):
            q = qsum[j] * (1.0 / L_WORD)
            rm = rtsum[j] * (1.0 / L_WORD) + rsum[j] * (1.0 / L_RELA)
            dv = dv + q * rm
            n1 = n1 + q * q
            n2 = n2 + rm * rm
        for s in (8, 4, 2, 1):
            idx = lane_iota ^ s
            dv = dv + dv.at[idx].get(mode="promise_in_bounds")
            n1 = n1 + n1.at[idx].get(mode="promise_in_bounds")
            n2 = n2 + n2.at[idx].get(mode="promise_in_bounds")
        return dv, n1, n2

    @pl.loop(0, BPW // LANES)
    def _group(g):
        def pbody(p, carry):
            dacc, n1acc, n2acc = carry
            for b in range(2):
                k = p * 2 + b
                e = g * LANES + k
                nxt = e + 1

                @pl.when(nxt < BPW)
                def _():
                    issue_w(jnp.minimum(nxt, BPW - 1), 1 - b)

                wait_w(e, b)
                d_v, n1_v, n2_v = compute_elem(wbufs[b], e)
                sel = lane_iota == k
                dacc = jnp.where(sel, d_v, dacc)
                n1acc = jnp.where(sel, n1_v, n1acc)
                n2acc = jnp.where(sel, n2_v, n2acc)
            return (dacc, n1acc, n2acc)

        zero = jnp.zeros((LANES,), jnp.float32)
        dacc, n1acc, n2acc = lax.fori_loop(0, LANES // 2, pbody,
                                           (zero, zero, zero))

        prod = n1acc * n2acc
        yi = (lax.bitcast_convert_type(prod, jnp.int32) >> 1) \
            + jnp.int32(0x1FBD1DF5)
        y = lax.bitcast_convert_type(yi, jnp.float32)
        for _ in range(3):
            y = 0.5 * (y + prod / y)
        score = dacc / jnp.maximum(y, EPS)
        score_v[pl.ds(g * LANES, LANES)] = score

    pltpu.sync_copy(score_v, out_hbm.at[pl.ds(base, BPW)])


@functools.cache
def _build(batch):
    mesh = plsc.VectorSubcoreMesh(core_axis_name="c", subcore_axis_name="s")
    return pl.kernel(
        _sc_body,
        out_type=jax.ShapeDtypeStruct((batch,), jnp.float32),
        mesh=mesh,
        compiler_params=pltpu.CompilerParams(use_tc_tiling_on_sc=False),
        scratch_types=[
            pltpu.VMEM((BPW, L_WORD), jnp.int32),
            pltpu.VMEM((BPW, L_WORD), jnp.int32),
            pltpu.VMEM((BPW * L_RELA + LANES,), jnp.int32),
            pltpu.VMEM((RELA_VOCAB, EMBED_DIM // 2), jnp.int32),
            pltpu.VMEM((WPAD, EMBED_DIM // 2), jnp.int32),
            pltpu.VMEM((WPAD, EMBED_DIM // 2), jnp.int32),
            pltpu.VMEM((BPW,), jnp.float32),
            pltpu.SemaphoreType.DMA,
            pltpu.SemaphoreType.DMA,
            pltpu.SemaphoreType.DMA,
            pltpu.SemaphoreType.DMA,
            pltpu.SemaphoreType.DMA,
        ],
    )


def kernel(ques_x, rela_text_x, rela_x, word_emb, rela_emb):
    batch = ques_x.shape[0]
    qidx = ques_x.astype(jnp.int32)
    tidx = rela_text_x.astype(jnp.int32)
    ridx = rela_x.astype(jnp.int32).reshape(-1)
    def pack(tab):
        x = lax.bitcast_convert_type(tab, jnp.int32) + jnp.int32(0x8000)
        half = tab.shape[1] // 2
        lo = lax.shift_right_logical(x[:, :half], 16)
        hi = x[:, half:] & jnp.int32(-65536)
        return lo | hi

    return _build(batch)(qidx, tidx, ridx, pack(word_emb), pack(rela_emb))


# speedup vs baseline: 13.5569x; 1.6015x over previous
import functools

import jax
import jax.numpy as jnp
from jax import lax
from jax.experimental import pallas as pl
from jax.experimental.pallas import tpu as pltpu
from jax.experimental.pallas import tpu_sc as plsc

EMBED_DIM = 128
L_WORD = 50
L_RELA = 8
WPAD = 100
EPS = 1e-8
LANES = 16
NCHUNK = EMBED_DIM // LANES
BPW = 128
RELA_VOCAB = 1000


def _rowsum_pair(wbuf, n):
    zero = jnp.zeros((LANES,), jnp.float32)

    def body(r, carry):
        qa, ra = carry
        qa = tuple(qa[j] + wbuf[r, pl.ds(j * LANES, LANES)] for j in range(NCHUNK))
        ra = tuple(ra[j] + wbuf[r + n, pl.ds(j * LANES, LANES)] for j in range(NCHUNK))
        return (qa, ra)

    init = (tuple(zero for _ in range(NCHUNK)), tuple(zero for _ in range(NCHUNK)))
    return lax.fori_loop(0, n, body, init, unroll=2)


def _sc_body(qidx_hbm, tidx_hbm, ridx_hbm, wtab, rtab, out_hbm,
             qidx_v, tidx_v, ridx_v, rtab_v, wb0, wb1, score_v,
             swa, swb, sta, stb, srt):
    nc = 2
    wid = lax.axis_index("s") * nc + lax.axis_index("c")
    base = wid * BPW

    pltpu.sync_copy(qidx_hbm.at[pl.ds(base, BPW)], qidx_v)
    pltpu.sync_copy(tidx_hbm.at[pl.ds(base, BPW)], tidx_v)
    pltpu.sync_copy(ridx_hbm.at[pl.ds(base * L_RELA, BPW * L_RELA)],
                    ridx_v.at[pl.ds(0, BPW * L_RELA)])
    pltpu.async_copy(rtab, rtab_v, srt)

    wbufs = (wb0, wb1)
    sws = (swa, swb)
    sts = (sta, stb)

    def issue_w(e, b):
        pltpu.async_copy(wtab.at[qidx_v.at[e]],
                         wbufs[b].at[pl.ds(0, L_WORD)], sws[b])
        pltpu.async_copy(wtab.at[tidx_v.at[e]],
                         wbufs[b].at[pl.ds(L_WORD, L_WORD)], sts[b])

    def wait_w(e, b):
        pltpu.make_async_copy(wtab.at[qidx_v.at[e]],
                              wbufs[b].at[pl.ds(0, L_WORD)], sws[b]).wait()
        pltpu.make_async_copy(wtab.at[tidx_v.at[e]],
                              wbufs[b].at[pl.ds(L_WORD, L_WORD)], sts[b]).wait()

    issue_w(0, 0)
    pltpu.make_async_copy(rtab, rtab_v, srt).wait()

    lane_iota = lax.iota(jnp.int32, LANES)

    def rela_sum(e):
        acc = [jnp.zeros((LANES,), jnp.float32) for _ in range(NCHUNK)]
        rv = ridx_v[pl.ds(e * L_RELA, LANES)]
        for i in range(L_RELA):
            r = rv[i]
            for c in range(NCHUNK // 2):
                x = rtab_v[r, pl.ds(c * LANES, LANES)]
                acc[2 * c] = acc[2 * c] + lax.bitcast_convert_type(
                    x << 16, jnp.float32)
                acc[2 * c + 1] = acc[2 * c + 1] + lax.bitcast_convert_type(
                    x & jnp.int32(-65536), jnp.float32)
        return acc

    def compute_elem(wbuf, rsum):
        qsum, rtsum = _rowsum_pair(wbuf, L_WORD)
        dv = jnp.zeros((LANES,), jnp.float32)
        n1 = jnp.zeros((LANES,), jnp.float32)
        n2 = jnp.zeros((LANES,), jnp.float32)
        for j in range(NCHUNK):
            q = qsum[j] * (1.0 / L_WORD)
            rm = rtsum[j] * (1.0 / L_WORD) + rsum[j] * (1.0 / L_RELA)
            dv = dv + q * rm
            n1 = n1 + q * q
            n2 = n2 + rm * rm
        for s in (8, 4, 2, 1):
            idx = lane_iota ^ s
            dv = dv + dv.at[idx].get(mode="promise_in_bounds")
            n1 = n1 + n1.at[idx].get(mode="promise_in_bounds")
            n2 = n2 + n2.at[idx].get(mode="promise_in_bounds")
        return dv, n1, n2

    @pl.loop(0, BPW // LANES)
    def _group(g):
        def pbody(p, carry):
            dacc, n1acc, n2acc = carry
            for b in range(2):
                k = p * 2 + b
                e = g * LANES + k
                nxt = e + 1

                @pl.when(nxt < BPW)
                def _():
                    issue_w(jnp.minimum(nxt, BPW - 1), 1 - b)

                rsum = rela_sum(e)
                wait_w(e, b)
                d_v, n1_v, n2_v = compute_elem(wbufs[b], rsum)
                sel = lane_iota == k
                dacc = jnp.where(sel, d_v, dacc)
                n1acc = jnp.where(sel, n1_v, n1acc)
                n2acc = jnp.where(sel, n2_v, n2acc)
            return (dacc, n1acc, n2acc)

        zero = jnp.zeros((LANES,), jnp.float32)
        dacc, n1acc, n2acc = lax.fori_loop(0, LANES // 2, pbody,
                                           (zero, zero, zero))

        prod = n1acc * n2acc
        yi = (lax.bitcast_convert_type(prod, jnp.int32) >> 1) \
            + jnp.int32(0x1FBD1DF5)
        y = lax.bitcast_convert_type(yi, jnp.float32)
        for _ in range(3):
            y = 0.5 * (y + prod / y)
        score = dacc / jnp.maximum(y, EPS)
        score_v[pl.ds(g * LANES, LANES)] = score

    pltpu.sync_copy(score_v, out_hbm.at[pl.ds(base, BPW)])


@functools.cache
def _build(batch):
    mesh = plsc.VectorSubcoreMesh(core_axis_name="c", subcore_axis_name="s")
    return pl.kernel(
        _sc_body,
        out_type=jax.ShapeDtypeStruct((batch,), jnp.float32),
        mesh=mesh,
        compiler_params=pltpu.CompilerParams(use_tc_tiling_on_sc=False),
        scratch_types=[
            pltpu.VMEM((BPW, L_WORD), jnp.int32),
            pltpu.VMEM((BPW, L_WORD), jnp.int32),
            pltpu.VMEM((BPW * L_RELA + LANES,), jnp.int32),
            pltpu.VMEM((RELA_VOCAB, EMBED_DIM // 2), jnp.int32),
            pltpu.VMEM((WPAD, EMBED_DIM), jnp.float32),
            pltpu.VMEM((WPAD, EMBED_DIM), jnp.float32),
            pltpu.VMEM((BPW,), jnp.float32),
            pltpu.SemaphoreType.DMA,
            pltpu.SemaphoreType.DMA,
            pltpu.SemaphoreType.DMA,
            pltpu.SemaphoreType.DMA,
            pltpu.SemaphoreType.DMA,
        ],
    )


def kernel(ques_x, rela_text_x, rela_x, word_emb, rela_emb):
    batch = ques_x.shape[0]
    qidx = ques_x.astype(jnp.int32)
    tidx = rela_text_x.astype(jnp.int32)
    ridx = rela_x.astype(jnp.int32).reshape(-1)
    v, d = rela_emb.shape
    rtab = lax.bitcast_convert_type(
        rela_emb.astype(jnp.bfloat16).reshape(v, d // 32, 2, 16)
        .transpose(0, 1, 3, 2), jnp.int32).reshape(v, d // 2)
    return _build(batch)(qidx, tidx, ridx, word_emb, rtab)
